# baseline (speedup 1.0000x reference)
import jax
import jax.numpy as jnp
from jax import lax
from jax.experimental import pallas as pl
from jax.experimental.pallas import tpu as pltpu
from jax.experimental.pallas import tpu_sc as plsc

N = 10000
E = 320000
D = 128
NT = 6
NC = 2
NS = 16
NPAD = 10240
ACC_R = 10112
CNT = 61440
CNT_T = CNT // NS
CHW = 80
ET = E // (NC * NS)
ETC = E // NS
NCHW = ET // CHW
CH = 16
NCH = ET // CH


def _bcast_lane(v16, r):
    idx = jnp.full((16, 1), r, dtype=jnp.int32)
    return lax.gather(
        v16, idx,
        dimension_numbers=lax.GatherDimensionNumbers(
            offset_dims=(), collapsed_slice_dims=(0,), start_index_map=(0,)),
        slice_sizes=(1,),
        mode=lax.GatherScatterMode.PROMISE_IN_BOUNDS)


def _cnt_body(dst_hbm, et_hbm, out_hbm, cnt_sh, dst1d, typ1d,
              ckey2d, ones_v, zbuf, wbuf, sem_c):
    c = lax.axis_index("c")
    s = lax.axis_index("s")

    z16f = jnp.zeros((16,), jnp.float32)
    o16f = jnp.ones((16,), jnp.float32)

    def fill_zero(i, carry):
        zbuf[pl.ds(i * 16, 16)] = z16f
        return carry
    lax.fori_loop(0, CNT_T // 16, fill_zero, 0)
    for g in range(CHW // 16):
        ones_v[pl.ds(g * 16, 16)] = o16f

    pltpu.sync_copy(zbuf.at[pl.ds(0, CNT_T)],
                    cnt_sh.at[pl.ds(s * CNT_T, CNT_T)])
    plsc.subcore_barrier()

    for h in range(2):
        eb0 = s * ETC + h * ET
        pltpu.sync_copy(dst_hbm.at[pl.ds(eb0, ET)], dst1d)
        pltpu.sync_copy(et_hbm.at[pl.ds(eb0, ET)], typ1d)

        def ckeys(g, carry):
            i = g // 5
            o = (g % 5) * 16
            d16 = dst1d[pl.ds(g * 16, 16)]
            t16 = typ1d[pl.ds(g * 16, 16)]
            ckey2d[i, pl.ds(o, 16)] = t16 * N + d16
            return carry
        lax.fori_loop(0, (ET // 16), ckeys, 0)

        def fire_cnt(i, carry):
            pltpu.async_copy(ones_v, cnt_sh.at[ckey2d.at[i]], sem_c, add=True)
            return carry
        lax.fori_loop(0, NCHW, fire_cnt, 0)

        def drain_cnt(i, carry):
            pltpu.make_async_copy(dst_hbm.at[pl.ds(0, CHW)],
                                  typ1d.at[pl.ds(0, CHW)], sem_c).wait()
            return carry
        lax.fori_loop(0, NCHW, drain_cnt, 0)

    plsc.subcore_barrier()

    eb = c * (E // NC) + s * ET
    pltpu.sync_copy(dst_hbm.at[pl.ds(eb, ET)], dst1d)
    pltpu.sync_copy(et_hbm.at[pl.ds(eb, ET)], typ1d)

    def wkeys(g, carry):
        i = g // 5
        o = (g % 5) * 16
        d16 = dst1d[pl.ds(g * 16, 16)]
        t16 = typ1d[pl.ds(g * 16, 16)]
        ckey2d[i, pl.ds(o, 16)] = t16 * N + d16
        return carry
    lax.fori_loop(0, (ET // 16), wkeys, 0)

    def fire_w(i, carry):
        pltpu.async_copy(cnt_sh.at[ckey2d.at[i]], wbuf.at[pl.ds(i * CHW, CHW)],
                         sem_c)
        return carry
    lax.fori_loop(0, NCHW, fire_w, 0)

    def drain_w(i, carry):
        pltpu.make_async_copy(dst_hbm.at[pl.ds(0, CHW)],
                              typ1d.at[pl.ds(0, CHW)], sem_c).wait()
        return carry
    lax.fori_loop(0, NCHW, drain_w, 0)

    def to_weight(g, carry):
        c16 = wbuf[pl.ds(g * 16, 16)]
        wbuf[pl.ds(g * 16, 16)] = 1.0 / jnp.maximum(c16, 1.0)
        return carry
    lax.fori_loop(0, (ET // 16), to_weight, 0)

    pltpu.sync_copy(wbuf, out_hbm.at[pl.ds(eb, ET)])


def _sc_counts(dst, edge_type):
    mesh = plsc.VectorSubcoreMesh(core_axis_name="c", subcore_axis_name="s")
    return pl.kernel(
        _cnt_body,
        out_type=jax.ShapeDtypeStruct((E,), jnp.float32),
        mesh=mesh,
        scratch_types=[
            pltpu.VMEM_SHARED((CNT,), jnp.float32),
            pltpu.VMEM((ET,), jnp.int32),
            pltpu.VMEM((ET,), jnp.int32),
            pltpu.VMEM((NCHW, CHW), jnp.int32),
            pltpu.VMEM((CHW,), jnp.float32),
            pltpu.VMEM((CNT_T,), jnp.float32),
            pltpu.VMEM((ET,), jnp.float32),
            pltpu.SemaphoreType.DMA,
        ],
    )(dst, edge_type)


def _agg_body(z_hbm, gk_hbm, dst_hbm, w_hbm, out_hbm,
              acc_sh,
              gb0, gb1, sb0, sb1, gk_v, w_v, dk2d,
              sg0, sg1, ss0, ss1, sm0, sm1):
    c = lax.axis_index("c")
    s = lax.axis_index("s")
    wid = c * NS + s
    e0 = wid * ET

    z16f = jnp.zeros((16,), jnp.float32)
    gbufs = (gb0, gb1)
    sbufs = (sb0, sb1)
    sgs = (sg0, sg1)
    sss = (ss0, ss1)
    sms = (sm0, sm1)

    def fill_zero(i, carry):
        for j in range(D // 16):
            sb0[i, pl.ds(j * 16, 16)] = z16f
        return carry
    lax.fori_loop(0, CHW, fill_zero, 0)

    ab = s * 632

    def zero_acc(k, carry):
        pltpu.sync_copy(sb0, acc_sh.at[pl.ds(ab + k * CHW, CHW)])
        return carry
    lax.fori_loop(0, 7, zero_acc, 0)
    pltpu.sync_copy(sb0.at[pl.ds(0, 72)], acc_sh.at[pl.ds(ab + 560, 72)])

    plsc.subcore_barrier()

    def fire_meta(i, m2, m4):
        eb = e0 + i * CHW
        pltpu.async_copy(gk_hbm.at[pl.ds(eb, CHW)], gk_v.at[m2], sms[m2])
        pltpu.async_copy(dst_hbm.at[pl.ds(eb, CHW)], dk2d.at[m4], sms[m2])
        pltpu.async_copy(w_hbm.at[pl.ds(eb, CHW)], w_v.at[m2], sms[m2])

    def drain_meta(m2, m4):
        pltpu.make_async_copy(gk_hbm.at[pl.ds(0, CHW)], gk_v.at[m2],
                              sms[m2]).wait()
        pltpu.make_async_copy(dst_hbm.at[pl.ds(0, CHW)], dk2d.at[m4],
                              sms[m2]).wait()
        pltpu.make_async_copy(w_hbm.at[pl.ds(0, CHW)], w_v.at[m2],
                              sms[m2]).wait()

    def fire_gather(b):
        pltpu.async_copy(z_hbm.at[gk_v.at[b]], gbufs[b], sgs[b])

    def wait_gather(b):
        pltpu.make_async_copy(z_hbm.at[pl.ds(0, CHW)], gbufs[b],
                              sgs[b]).wait()

    def fire_scatter(b, m4):
        pltpu.async_copy(sbufs[b], acc_sh.at[dk2d.at[m4]], sss[b], add=True)

    def wait_scatter(b):
        pltpu.make_async_copy(z_hbm.at[pl.ds(0, CHW)], sbufs[b],
                              sss[b]).wait()

    def scale(b):
        def grp(g, carry):
            w16 = w_v[b, pl.ds(g * 16, 16)]

            def row_body(r, cc):
                wr = _bcast_lane(w16, r)
                row = g * 16 + r
                for jj in range(D // 16):
                    sbufs[b][row, pl.ds(jj * 16, 16)] = (
                        gbufs[b][row, pl.ds(jj * 16, 16)] * wr)
                return cc
            lax.fori_loop(0, 16, row_body, 0)
            return carry
        lax.fori_loop(0, CHW // 16, grp, 0)

    fire_meta(0, 0, 0)
    fire_meta(1, 1, 1)
    drain_meta(0, 0)
    fire_gather(0)

    drain_meta(1, 1)
    fire_gather(1)
    wait_gather(0)
    scale(0)
    fire_scatter(0, 0)
    fire_meta(2, 0, 2)

    drain_meta(0, 2)
    fire_gather(0)
    wait_gather(1)
    scale(1)
    fire_scatter(1, 1)
    fire_meta(3, 1, 3)

    def step(k, carry):
        for b in range(4):
            i = 4 * k + b + 2
            drain_meta((b + 1) % 2, (b + 3) % 4)
            fire_gather((b + 1) % 2)
            wait_gather(b % 2)
            wait_scatter(b % 2)
            scale(b % 2)
            fire_scatter(b % 2, (b + 2) % 4)
            fire_meta(i + 2, b % 2, b)
        return carry
    lax.fori_loop(0, 30, step, 0)

    drain_meta(1, 3)
    fire_gather(1)
    wait_gather(0)
    wait_scatter(0)
    scale(0)
    fire_scatter(0, 2)
    fire_meta(124, 0, 0)

    drain_meta(0, 0)
    fire_gather(0)
    wait_gather(1)
    wait_scatter(1)
    scale(1)
    fire_scatter(1, 3)

    wait_gather(0)
    wait_scatter(0)
    scale(0)
    fire_scatter(0, 0)

    wait_scatter(1)
    wait_scatter(0)

    plsc.subcore_barrier()

    def flush(k, carry):
        ro = ab + k * CHW
        pltpu.sync_copy(acc_sh.at[pl.ds(ro, CHW)], out_hbm.at[c, pl.ds(ro, CHW)])
        return carry
    lax.fori_loop(0, 7, flush, 0)
    pltpu.sync_copy(acc_sh.at[pl.ds(ab + 560, 72)],
                    out_hbm.at[c, pl.ds(ab + 560, 72)])


def _sc_aggregate(z, gkey, dst, w):
    mesh = plsc.VectorSubcoreMesh(core_axis_name="c", subcore_axis_name="s")
    return pl.kernel(
        _agg_body,
        out_type=jax.ShapeDtypeStruct((NC, NPAD, D), jnp.float32),
        mesh=mesh,
        scratch_types=[
            pltpu.VMEM_SHARED((ACC_R, D), jnp.float32),
            pltpu.VMEM((CHW, D), jnp.float32),
            pltpu.VMEM((CHW, D), jnp.float32),
            pltpu.VMEM((CHW, D), jnp.float32),
            pltpu.VMEM((CHW, D), jnp.float32),
            pltpu.VMEM((2, CHW), jnp.int32),
            pltpu.VMEM((2, CHW), jnp.float32),
            pltpu.VMEM((4, CHW), jnp.int32),
            pltpu.SemaphoreType.DMA,
            pltpu.SemaphoreType.DMA,
            pltpu.SemaphoreType.DMA,
            pltpu.SemaphoreType.DMA,
            pltpu.SemaphoreType.DMA,
            pltpu.SemaphoreType.DMA,
        ],
    )(z, gkey, dst, w)


def _mm_body(x_ref, w_ref, o_ref):
    o_ref[0] = jnp.dot(x_ref[...], w_ref[0],
                       preferred_element_type=jnp.float32)


def _tc_pretransform(x, W_l):
    blk = 1000
    z3 = pl.pallas_call(
        _mm_body,
        grid=(N // blk, NT),
        in_specs=[
            pl.BlockSpec((blk, D), lambda j, t: (j, 0)),
            pl.BlockSpec((1, D, D), lambda j, t: (t, 0, 0)),
        ],
        out_specs=pl.BlockSpec((1, blk, D), lambda j, t: (t, j, 0)),
        out_shape=jax.ShapeDtypeStruct((NT, N, D), jnp.float32),
    )(x, W_l)
    return z3.reshape(NT * N, D)


def _gk_body(s_ref, t_ref, o_ref):
    o_ref[...] = t_ref[...] * N + s_ref[...]


def _tc_gkey(src, edge_type):
    g2 = pl.pallas_call(
        _gk_body,
        grid=(1,),
        in_specs=[
            pl.BlockSpec((E // D, D), lambda j: (0, 0)),
            pl.BlockSpec((E // D, D), lambda j: (0, 0)),
        ],
        out_specs=pl.BlockSpec((E // D, D), lambda j: (0, 0)),
        out_shape=jax.ShapeDtypeStruct((E // D, D), jnp.int32),
    )(src.reshape(E // D, D), edge_type.reshape(E // D, D))
    return g2.reshape(E)


def _fin_body(s_ref, x_ref, wr_ref, b_ref, g_ref, be_ref, o_ref):
    h = (s_ref[0] + s_ref[1]
         + jnp.dot(x_ref[...], wr_ref[...], preferred_element_type=jnp.float32)
         + b_ref[...])
    mu = jnp.mean(h, axis=-1, keepdims=True)
    d = h - mu
    var = jnp.mean(d * d, axis=-1, keepdims=True)
    y = d * lax.rsqrt(var + 1e-5) * g_ref[...] + be_ref[...]
    o_ref[...] = jnp.maximum(y, 0.0)


def _tc_finish(S, x_pad, wr_sum, bias, gamma, beta):
    blk = 1024
    return pl.pallas_call(
        _fin_body,
        grid=(NPAD // blk,),
        in_specs=[
            pl.BlockSpec((NC, blk, D), lambda j: (0, j, 0)),
            pl.BlockSpec((blk, D), lambda j: (j, 0)),
            pl.BlockSpec((D, D), lambda j: (0, 0)),
            pl.BlockSpec((1, D), lambda j: (0, 0)),
            pl.BlockSpec((1, D), lambda j: (0, 0)),
            pl.BlockSpec((1, D), lambda j: (0, 0)),
        ],
        out_specs=pl.BlockSpec((blk, D), lambda j: (j, 0)),
        out_shape=jax.ShapeDtypeStruct((NPAD, D), jnp.float32),
    )(S, x_pad, wr_sum, bias, gamma, beta)


@jax.jit
def kernel(x, edge_index, edge_type, W_l, b_l, W_r, emb, gamma, beta):
    z = _tc_pretransform(x, W_l)
    gkey = _tc_gkey(edge_index[0], edge_type)
    w = _sc_counts(edge_index[1], edge_type)
    S = _sc_aggregate(z, gkey, edge_index[1], w)
    wr_sum = jnp.sum(W_r, axis=0)
    bias = (jnp.sum(b_l, axis=0) + jnp.sum(emb, axis=0)).reshape(1, D)
    x_pad = jnp.pad(x, ((0, NPAD - N), (0, 0)))
    out = _tc_finish(S, x_pad, wr_sum, bias,
                     gamma.reshape(1, D), beta.reshape(1, D))
    return out[:N]


# speedup vs baseline: 1.0036x; 1.0036x over previous
import jax
import jax.numpy as jnp
from jax import lax
from jax.experimental import pallas as pl
from jax.experimental.pallas import tpu as pltpu
from jax.experimental.pallas import tpu_sc as plsc

N = 10000
E = 320000
D = 128
NT = 6
NC = 2
NS = 16
NPAD = 10240
ACC_R = 10112
CNT = 61440
CNT_T = CNT // NS
CHW = 80
ET = E // (NC * NS)
ETC = E // NS
NCHW = ET // CHW
CH = 16
NCH = ET // CH


def _bcast_lane(v16, r):
    idx = jnp.full((16, 1), r, dtype=jnp.int32)
    return lax.gather(
        v16, idx,
        dimension_numbers=lax.GatherDimensionNumbers(
            offset_dims=(), collapsed_slice_dims=(0,), start_index_map=(0,)),
        slice_sizes=(1,),
        mode=lax.GatherScatterMode.PROMISE_IN_BOUNDS)


def _cnt_body(dst_hbm, et_hbm, out_hbm, cnt_sh, dst1d, typ1d,
              ckey2d, ones_v, zbuf, wbuf, sem_c):
    c = lax.axis_index("c")
    s = lax.axis_index("s")

    z16f = jnp.zeros((16,), jnp.float32)
    o16f = jnp.ones((16,), jnp.float32)

    def fill_zero(i, carry):
        zbuf[pl.ds(i * 16, 16)] = z16f
        return carry
    lax.fori_loop(0, CNT_T // 16, fill_zero, 0)
    for g in range(CHW // 16):
        ones_v[pl.ds(g * 16, 16)] = o16f

    pltpu.sync_copy(zbuf.at[pl.ds(0, CNT_T)],
                    cnt_sh.at[pl.ds(s * CNT_T, CNT_T)])
    plsc.subcore_barrier()

    for h in range(2):
        eb0 = s * ETC + h * ET
        pltpu.sync_copy(dst_hbm.at[pl.ds(eb0, ET)], dst1d)
        pltpu.sync_copy(et_hbm.at[pl.ds(eb0, ET)], typ1d)

        def ckeys(g, carry):
            i = g // 5
            o = (g % 5) * 16
            d16 = dst1d[pl.ds(g * 16, 16)]
            t16 = typ1d[pl.ds(g * 16, 16)]
            ckey2d[i, pl.ds(o, 16)] = t16 * N + d16
            return carry
        lax.fori_loop(0, (ET // 16), ckeys, 0)

        def fire_cnt(i, carry):
            pltpu.async_copy(ones_v, cnt_sh.at[ckey2d.at[i]], sem_c, add=True)
            return carry
        lax.fori_loop(0, NCHW, fire_cnt, 0)

        def drain_cnt(i, carry):
            pltpu.make_async_copy(dst_hbm.at[pl.ds(0, CHW)],
                                  typ1d.at[pl.ds(0, CHW)], sem_c).wait()
            return carry
        lax.fori_loop(0, NCHW, drain_cnt, 0)

    plsc.subcore_barrier()

    eb = c * (E // NC) + s * ET
    pltpu.sync_copy(dst_hbm.at[pl.ds(eb, ET)], dst1d)
    pltpu.sync_copy(et_hbm.at[pl.ds(eb, ET)], typ1d)

    def wkeys(g, carry):
        i = g // 5
        o = (g % 5) * 16
        d16 = dst1d[pl.ds(g * 16, 16)]
        t16 = typ1d[pl.ds(g * 16, 16)]
        ckey2d[i, pl.ds(o, 16)] = t16 * N + d16
        return carry
    lax.fori_loop(0, (ET // 16), wkeys, 0)

    def fire_w(i, carry):
        pltpu.async_copy(cnt_sh.at[ckey2d.at[i]], wbuf.at[pl.ds(i * CHW, CHW)],
                         sem_c)
        return carry
    lax.fori_loop(0, NCHW, fire_w, 0)

    def drain_w(i, carry):
        pltpu.make_async_copy(dst_hbm.at[pl.ds(0, CHW)],
                              typ1d.at[pl.ds(0, CHW)], sem_c).wait()
        return carry
    lax.fori_loop(0, NCHW, drain_w, 0)

    def to_weight(g, carry):
        c16 = wbuf[pl.ds(g * 16, 16)]
        wbuf[pl.ds(g * 16, 16)] = 1.0 / jnp.maximum(c16, 1.0)
        return carry
    lax.fori_loop(0, (ET // 16), to_weight, 0)

    pltpu.sync_copy(wbuf, out_hbm.at[pl.ds(eb, ET)])


def _sc_counts(dst, edge_type):
    mesh = plsc.VectorSubcoreMesh(core_axis_name="c", subcore_axis_name="s")
    return pl.kernel(
        _cnt_body,
        out_type=jax.ShapeDtypeStruct((E,), jnp.float32),
        mesh=mesh,
        scratch_types=[
            pltpu.VMEM_SHARED((CNT,), jnp.float32),
            pltpu.VMEM((ET,), jnp.int32),
            pltpu.VMEM((ET,), jnp.int32),
            pltpu.VMEM((NCHW, CHW), jnp.int32),
            pltpu.VMEM((CHW,), jnp.float32),
            pltpu.VMEM((CNT_T,), jnp.float32),
            pltpu.VMEM((ET,), jnp.float32),
            pltpu.SemaphoreType.DMA,
        ],
    )(dst, edge_type)


def _agg_body(z_hbm, gk_hbm, dst_hbm, w_hbm, out_hbm,
              acc_sh,
              gb0, gb1, sb0, sb1, gk_v, w_v, dk2d,
              sg0, sg1, ss0, ss1, sm0, sm1):
    c = lax.axis_index("c")
    s = lax.axis_index("s")
    wid = c * NS + s
    e0 = wid * ET

    z16f = jnp.zeros((16,), jnp.float32)
    gbufs = (gb0, gb1)
    sbufs = (sb0, sb1)
    sgs = (sg0, sg1)
    sss = (ss0, ss1)
    sms = (sm0, sm1)

    def fill_zero(i, carry):
        for j in range(D // 16):
            sb0[i, pl.ds(j * 16, 16)] = z16f
        return carry
    lax.fori_loop(0, CHW, fill_zero, 0)

    ab = s * 632

    def zero_acc(k, carry):
        pltpu.sync_copy(sb0, acc_sh.at[pl.ds(ab + k * CHW, CHW)])
        return carry
    lax.fori_loop(0, 7, zero_acc, 0)
    pltpu.sync_copy(sb0.at[pl.ds(0, 72)], acc_sh.at[pl.ds(ab + 560, 72)])

    plsc.subcore_barrier()

    def fire_meta(i, m2, m4):
        eb = e0 + i * CHW
        pltpu.async_copy(gk_hbm.at[pl.ds(eb, CHW)], gk_v.at[m2], sms[m2])
        pltpu.async_copy(dst_hbm.at[pl.ds(eb, CHW)], dk2d.at[m4], sms[m2])
        pltpu.async_copy(w_hbm.at[pl.ds(eb, CHW)], w_v.at[m2], sms[m2])

    def drain_meta(m2, m4):
        pltpu.make_async_copy(gk_hbm.at[pl.ds(0, CHW)], gk_v.at[m2],
                              sms[m2]).wait()
        pltpu.make_async_copy(dst_hbm.at[pl.ds(0, CHW)], dk2d.at[m4],
                              sms[m2]).wait()
        pltpu.make_async_copy(w_hbm.at[pl.ds(0, CHW)], w_v.at[m2],
                              sms[m2]).wait()

    def fire_gather(b):
        pltpu.async_copy(z_hbm.at[gk_v.at[b]], gbufs[b], sgs[b])

    def wait_gather(b):
        pltpu.make_async_copy(z_hbm.at[pl.ds(0, CHW)], gbufs[b],
                              sgs[b]).wait()

    def fire_scatter(b, m4):
        pltpu.async_copy(sbufs[b], acc_sh.at[dk2d.at[m4]], sss[b], add=True)

    def wait_scatter(b):
        pltpu.make_async_copy(z_hbm.at[pl.ds(0, CHW)], sbufs[b],
                              sss[b]).wait()

    def scale(b):
        def grp(g, carry):
            w16 = w_v[b, pl.ds(g * 16, 16)]

            def row_body(rq, cc):
                for u in range(4):
                    r = rq * 4 + u
                    wr = _bcast_lane(w16, r)
                    row = g * 16 + r
                    for jj in range(D // 16):
                        sbufs[b][row, pl.ds(jj * 16, 16)] = (
                            gbufs[b][row, pl.ds(jj * 16, 16)] * wr)
                return cc
            lax.fori_loop(0, 4, row_body, 0)
            return carry
        lax.fori_loop(0, CHW // 16, grp, 0)

    fire_meta(0, 0, 0)
    fire_meta(1, 1, 1)
    drain_meta(0, 0)
    fire_gather(0)

    drain_meta(1, 1)
    fire_gather(1)
    wait_gather(0)
    scale(0)
    fire_scatter(0, 0)
    fire_meta(2, 0, 2)

    drain_meta(0, 2)
    fire_gather(0)
    wait_gather(1)
    scale(1)
    fire_scatter(1, 1)
    fire_meta(3, 1, 3)

    def step(k, carry):
        for b in range(4):
            i = 4 * k + b + 2
            drain_meta((b + 1) % 2, (b + 3) % 4)
            fire_gather((b + 1) % 2)
            wait_gather(b % 2)
            wait_scatter(b % 2)
            scale(b % 2)
            fire_scatter(b % 2, (b + 2) % 4)
            fire_meta(i + 2, b % 2, b)
        return carry
    lax.fori_loop(0, 30, step, 0)

    drain_meta(1, 3)
    fire_gather(1)
    wait_gather(0)
    wait_scatter(0)
    scale(0)
    fire_scatter(0, 2)
    fire_meta(124, 0, 0)

    drain_meta(0, 0)
    fire_gather(0)
    wait_gather(1)
    wait_scatter(1)
    scale(1)
    fire_scatter(1, 3)

    wait_gather(0)
    wait_scatter(0)
    scale(0)
    fire_scatter(0, 0)

    wait_scatter(1)
    wait_scatter(0)

    plsc.subcore_barrier()

    def flush(k, carry):
        ro = ab + k * CHW
        pltpu.sync_copy(acc_sh.at[pl.ds(ro, CHW)], out_hbm.at[c, pl.ds(ro, CHW)])
        return carry
    lax.fori_loop(0, 7, flush, 0)
    pltpu.sync_copy(acc_sh.at[pl.ds(ab + 560, 72)],
                    out_hbm.at[c, pl.ds(ab + 560, 72)])


def _sc_aggregate(z, gkey, dst, w):
    mesh = plsc.VectorSubcoreMesh(core_axis_name="c", subcore_axis_name="s")
    return pl.kernel(
        _agg_body,
        out_type=jax.ShapeDtypeStruct((NC, NPAD, D), jnp.float32),
        mesh=mesh,
        scratch_types=[
            pltpu.VMEM_SHARED((ACC_R, D), jnp.float32),
            pltpu.VMEM((CHW, D), jnp.float32),
            pltpu.VMEM((CHW, D), jnp.float32),
            pltpu.VMEM((CHW, D), jnp.float32),
            pltpu.VMEM((CHW, D), jnp.float32),
            pltpu.VMEM((2, CHW), jnp.int32),
            pltpu.VMEM((2, CHW), jnp.float32),
            pltpu.VMEM((4, CHW), jnp.int32),
            pltpu.SemaphoreType.DMA,
            pltpu.SemaphoreType.DMA,
            pltpu.SemaphoreType.DMA,
            pltpu.SemaphoreType.DMA,
            pltpu.SemaphoreType.DMA,
            pltpu.SemaphoreType.DMA,
        ],
    )(z, gkey, dst, w)


def _mm_body(x_ref, w_ref, o_ref):
    o_ref[0] = jnp.dot(x_ref[...], w_ref[0],
                       preferred_element_type=jnp.float32)


def _tc_pretransform(x, W_l):
    blk = 1000
    z3 = pl.pallas_call(
        _mm_body,
        grid=(N // blk, NT),
        in_specs=[
            pl.BlockSpec((blk, D), lambda j, t: (j, 0)),
            pl.BlockSpec((1, D, D), lambda j, t: (t, 0, 0)),
        ],
        out_specs=pl.BlockSpec((1, blk, D), lambda j, t: (t, j, 0)),
        out_shape=jax.ShapeDtypeStruct((NT, N, D), jnp.float32),
    )(x, W_l)
    return z3.reshape(NT * N, D)


def _gk_body(s_ref, t_ref, o_ref):
    o_ref[...] = t_ref[...] * N + s_ref[...]


def _tc_gkey(src, edge_type):
    g2 = pl.pallas_call(
        _gk_body,
        grid=(1,),
        in_specs=[
            pl.BlockSpec((E // D, D), lambda j: (0, 0)),
            pl.BlockSpec((E // D, D), lambda j: (0, 0)),
        ],
        out_specs=pl.BlockSpec((E // D, D), lambda j: (0, 0)),
        out_shape=jax.ShapeDtypeStruct((E // D, D), jnp.int32),
    )(src.reshape(E // D, D), edge_type.reshape(E // D, D))
    return g2.reshape(E)


def _fin_body(s_ref, x_ref, wr_ref, b_ref, g_ref, be_ref, o_ref):
    h = (s_ref[0] + s_ref[1]
         + jnp.dot(x_ref[...], wr_ref[...], preferred_element_type=jnp.float32)
         + b_ref[...])
    mu = jnp.mean(h, axis=-1, keepdims=True)
    d = h - mu
    var = jnp.mean(d * d, axis=-1, keepdims=True)
    y = d * lax.rsqrt(var + 1e-5) * g_ref[...] + be_ref[...]
    o_ref[...] = jnp.maximum(y, 0.0)


def _tc_finish(S, x_pad, wr_sum, bias, gamma, beta):
    blk = 1024
    return pl.pallas_call(
        _fin_body,
        grid=(NPAD // blk,),
        in_specs=[
            pl.BlockSpec((NC, blk, D), lambda j: (0, j, 0)),
            pl.BlockSpec((blk, D), lambda j: (j, 0)),
            pl.BlockSpec((D, D), lambda j: (0, 0)),
            pl.BlockSpec((1, D), lambda j: (0, 0)),
            pl.BlockSpec((1, D), lambda j: (0, 0)),
            pl.BlockSpec((1, D), lambda j: (0, 0)),
        ],
        out_specs=pl.BlockSpec((blk, D), lambda j: (j, 0)),
        out_shape=jax.ShapeDtypeStruct((NPAD, D), jnp.float32),
    )(S, x_pad, wr_sum, bias, gamma, beta)


@jax.jit
def kernel(x, edge_index, edge_type, W_l, b_l, W_r, emb, gamma, beta):
    z = _tc_pretransform(x, W_l)
    gkey = _tc_gkey(edge_index[0], edge_type)
    w = _sc_counts(edge_index[1], edge_type)
    S = _sc_aggregate(z, gkey, edge_index[1], w)
    wr_sum = jnp.sum(W_r, axis=0)
    bias = (jnp.sum(b_l, axis=0) + jnp.sum(emb, axis=0)).reshape(1, D)
    x_pad = jnp.pad(x, ((0, NPAD - N), (0, 0)))
    out = _tc_finish(S, x_pad, wr_sum, bias,
                     gamma.reshape(1, D), beta.reshape(1, D))
    return out[:N]


# speedup vs baseline: 1.2797x; 1.2751x over previous
import jax
import jax.numpy as jnp
from jax import lax
from jax.experimental import pallas as pl
from jax.experimental.pallas import tpu as pltpu
from jax.experimental.pallas import tpu_sc as plsc

N = 10000
E = 320000
D = 128
NT = 6
NC = 2
NS = 16
NPAD = 10240
ACC_R = 10112
CNT = 61440
CNT_T = CNT // NS
CHW = 80
ET = E // (NC * NS)
ETC = E // NS
NCHW = ET // CHW
CH = 16
NCH = ET // CH


def _bcast_lane(v16, r):
    idx = jnp.full((16, 1), r, dtype=jnp.int32)
    return lax.gather(
        v16, idx,
        dimension_numbers=lax.GatherDimensionNumbers(
            offset_dims=(), collapsed_slice_dims=(0,), start_index_map=(0,)),
        slice_sizes=(1,),
        mode=lax.GatherScatterMode.PROMISE_IN_BOUNDS)


def _cnt_body(dst_hbm, et_hbm, out_hbm, cnt_sh, dst1d, typ1d,
              ckey2d, ones_v, zbuf, wbuf, sem_c):
    c = lax.axis_index("c")
    s = lax.axis_index("s")

    z16f = jnp.zeros((16,), jnp.float32)
    o16f = jnp.ones((16,), jnp.float32)

    def fill_zero(i, carry):
        zbuf[pl.ds(i * 16, 16)] = z16f
        return carry
    lax.fori_loop(0, CNT_T // 16, fill_zero, 0)
    for g in range(CHW // 16):
        ones_v[pl.ds(g * 16, 16)] = o16f

    pltpu.sync_copy(zbuf.at[pl.ds(0, CNT_T)],
                    cnt_sh.at[pl.ds(s * CNT_T, CNT_T)])
    plsc.subcore_barrier()

    for h in range(2):
        eb0 = s * ETC + h * ET
        pltpu.sync_copy(dst_hbm.at[pl.ds(eb0, ET)], dst1d)
        pltpu.sync_copy(et_hbm.at[pl.ds(eb0, ET)], typ1d)

        def ckeys(g, carry):
            i = g // 5
            o = (g % 5) * 16
            d16 = dst1d[pl.ds(g * 16, 16)]
            t16 = typ1d[pl.ds(g * 16, 16)]
            ckey2d[i, pl.ds(o, 16)] = t16 * N + d16
            return carry
        lax.fori_loop(0, (ET // 16), ckeys, 0)

        def fire_cnt(i, carry):
            pltpu.async_copy(ones_v, cnt_sh.at[ckey2d.at[i]], sem_c, add=True)
            return carry
        lax.fori_loop(0, NCHW, fire_cnt, 0)

        def drain_cnt(i, carry):
            pltpu.make_async_copy(dst_hbm.at[pl.ds(0, CHW)],
                                  typ1d.at[pl.ds(0, CHW)], sem_c).wait()
            return carry
        lax.fori_loop(0, NCHW, drain_cnt, 0)

    plsc.subcore_barrier()

    eb = c * (E // NC) + s * ET
    pltpu.sync_copy(dst_hbm.at[pl.ds(eb, ET)], dst1d)
    pltpu.sync_copy(et_hbm.at[pl.ds(eb, ET)], typ1d)

    def wkeys(g, carry):
        i = g // 5
        o = (g % 5) * 16
        d16 = dst1d[pl.ds(g * 16, 16)]
        t16 = typ1d[pl.ds(g * 16, 16)]
        ckey2d[i, pl.ds(o, 16)] = t16 * N + d16
        return carry
    lax.fori_loop(0, (ET // 16), wkeys, 0)

    def fire_w(i, carry):
        pltpu.async_copy(cnt_sh.at[ckey2d.at[i]], wbuf.at[pl.ds(i * CHW, CHW)],
                         sem_c)
        return carry
    lax.fori_loop(0, NCHW, fire_w, 0)

    def drain_w(i, carry):
        pltpu.make_async_copy(dst_hbm.at[pl.ds(0, CHW)],
                              typ1d.at[pl.ds(0, CHW)], sem_c).wait()
        return carry
    lax.fori_loop(0, NCHW, drain_w, 0)

    def to_weight(g, carry):
        c16 = wbuf[pl.ds(g * 16, 16)]
        wbuf[pl.ds(g * 16, 16)] = 1.0 / jnp.maximum(c16, 1.0)
        return carry
    lax.fori_loop(0, (ET // 16), to_weight, 0)

    pltpu.sync_copy(wbuf, out_hbm.at[pl.ds(eb, ET)])


def _sc_counts(dst, edge_type):
    mesh = plsc.VectorSubcoreMesh(core_axis_name="c", subcore_axis_name="s")
    return pl.kernel(
        _cnt_body,
        out_type=jax.ShapeDtypeStruct((E,), jnp.float32),
        mesh=mesh,
        scratch_types=[
            pltpu.VMEM_SHARED((CNT,), jnp.float32),
            pltpu.VMEM((ET,), jnp.int32),
            pltpu.VMEM((ET,), jnp.int32),
            pltpu.VMEM((NCHW, CHW), jnp.int32),
            pltpu.VMEM((CHW,), jnp.float32),
            pltpu.VMEM((CNT_T,), jnp.float32),
            pltpu.VMEM((ET,), jnp.float32),
            pltpu.SemaphoreType.DMA,
        ],
    )(dst, edge_type)


def _agg_body(z_hbm, gk_hbm, dst_hbm, w_hbm, out_hbm,
              acc_sh,
              gk_v, dk_v, w_v, rows_v, gk3, dk3, w3, rows3, zero_v):
    c = lax.axis_index("c")
    s = lax.axis_index("s")
    wid = c * NS + s
    e0 = wid * ET

    z16f = jnp.zeros((16,), jnp.float32)

    def fill_zero(i, carry):
        for j in range(D // 16):
            zero_v[i, pl.ds(j * 16, 16)] = z16f
        return carry
    lax.fori_loop(0, CHW, fill_zero, 0)

    ab = s * 632

    def zero_acc(k, carry):
        pltpu.sync_copy(zero_v, acc_sh.at[pl.ds(ab + k * CHW, CHW)])
        return carry
    lax.fori_loop(0, 7, zero_acc, 0)
    pltpu.sync_copy(zero_v.at[pl.ds(0, 72)], acc_sh.at[pl.ds(ab + 560, 72)])

    plsc.subcore_barrier()

    def agg(i, carry):
        eb = e0 + i * 128
        pltpu.sync_copy(gk_hbm.at[pl.ds(eb, 128)], gk_v.at[0])
        pltpu.sync_copy(dst_hbm.at[pl.ds(eb, 128)], dk_v.at[0])
        pltpu.sync_copy(w_hbm.at[pl.ds(eb, 128)], w_v)
        pltpu.sync_copy(z_hbm.at[gk_v.at[0]], rows_v)
        for g in range(128 // 16):
            w16 = w_v[pl.ds(g * 16, 16)]
            for r in range(16):
                wr = _bcast_lane(w16, r)
                row = g * 16 + r
                for jj in range(D // 16):
                    rows_v[row, pl.ds(jj * 16, 16)] = (
                        rows_v[row, pl.ds(jj * 16, 16)] * wr)
        pltpu.sync_copy(rows_v, acc_sh.at[dk_v.at[0]], add=True)
        return carry
    lax.fori_loop(0, 78, agg, 0)

    eb = e0 + 9984
    pltpu.sync_copy(gk_hbm.at[pl.ds(eb, 16)], gk3.at[0])
    pltpu.sync_copy(dst_hbm.at[pl.ds(eb, 16)], dk3.at[0])
    pltpu.sync_copy(w_hbm.at[pl.ds(eb, 16)], w3)
    pltpu.sync_copy(z_hbm.at[gk3.at[0]], rows3)
    w16 = w3[pl.ds(0, 16)]
    for r in range(16):
        wr = _bcast_lane(w16, r)
        for jj in range(D // 16):
            rows3[r, pl.ds(jj * 16, 16)] = rows3[r, pl.ds(jj * 16, 16)] * wr
    pltpu.sync_copy(rows3, acc_sh.at[dk3.at[0]], add=True)

    plsc.subcore_barrier()

    def flush(k, carry):
        ro = ab + k * CHW
        pltpu.sync_copy(acc_sh.at[pl.ds(ro, CHW)], out_hbm.at[c, pl.ds(ro, CHW)])
        return carry
    lax.fori_loop(0, 7, flush, 0)
    pltpu.sync_copy(acc_sh.at[pl.ds(ab + 560, 72)],
                    out_hbm.at[c, pl.ds(ab + 560, 72)])


def _sc_aggregate(z, gkey, dst, w):
    mesh = plsc.VectorSubcoreMesh(core_axis_name="c", subcore_axis_name="s")
    return pl.kernel(
        _agg_body,
        out_type=jax.ShapeDtypeStruct((NC, NPAD, D), jnp.float32),
        mesh=mesh,
        scratch_types=[
            pltpu.VMEM_SHARED((ACC_R, D), jnp.float32),
            pltpu.VMEM((1, 128), jnp.int32),
            pltpu.VMEM((1, 128), jnp.int32),
            pltpu.VMEM((128,), jnp.float32),
            pltpu.VMEM((128, D), jnp.float32),
            pltpu.VMEM((1, 16), jnp.int32),
            pltpu.VMEM((1, 16), jnp.int32),
            pltpu.VMEM((16,), jnp.float32),
            pltpu.VMEM((16, D), jnp.float32),
            pltpu.VMEM((CHW, D), jnp.float32),
        ],
    )(z, gkey, dst, w)


def _mm_body(x_ref, w_ref, o_ref):
    o_ref[0] = jnp.dot(x_ref[...], w_ref[0],
                       preferred_element_type=jnp.float32)


def _tc_pretransform(x, W_l):
    blk = 1000
    z3 = pl.pallas_call(
        _mm_body,
        grid=(N // blk, NT),
        in_specs=[
            pl.BlockSpec((blk, D), lambda j, t: (j, 0)),
            pl.BlockSpec((1, D, D), lambda j, t: (t, 0, 0)),
        ],
        out_specs=pl.BlockSpec((1, blk, D), lambda j, t: (t, j, 0)),
        out_shape=jax.ShapeDtypeStruct((NT, N, D), jnp.float32),
    )(x, W_l)
    return z3.reshape(NT * N, D)


def _gk_body(s_ref, t_ref, o_ref):
    o_ref[...] = t_ref[...] * N + s_ref[...]


def _tc_gkey(src, edge_type):
    g2 = pl.pallas_call(
        _gk_body,
        grid=(1,),
        in_specs=[
            pl.BlockSpec((E // D, D), lambda j: (0, 0)),
            pl.BlockSpec((E // D, D), lambda j: (0, 0)),
        ],
        out_specs=pl.BlockSpec((E // D, D), lambda j: (0, 0)),
        out_shape=jax.ShapeDtypeStruct((E // D, D), jnp.int32),
    )(src.reshape(E // D, D), edge_type.reshape(E // D, D))
    return g2.reshape(E)


def _fin_body(s_ref, x_ref, wr_ref, b_ref, g_ref, be_ref, o_ref):
    h = (s_ref[0] + s_ref[1]
         + jnp.dot(x_ref[...], wr_ref[...], preferred_element_type=jnp.float32)
         + b_ref[...])
    mu = jnp.mean(h, axis=-1, keepdims=True)
    d = h - mu
    var = jnp.mean(d * d, axis=-1, keepdims=True)
    y = d * lax.rsqrt(var + 1e-5) * g_ref[...] + be_ref[...]
    o_ref[...] = jnp.maximum(y, 0.0)


def _tc_finish(S, x_pad, wr_sum, bias, gamma, beta):
    blk = 1024
    return pl.pallas_call(
        _fin_body,
        grid=(NPAD // blk,),
        in_specs=[
            pl.BlockSpec((NC, blk, D), lambda j: (0, j, 0)),
            pl.BlockSpec((blk, D), lambda j: (j, 0)),
            pl.BlockSpec((D, D), lambda j: (0, 0)),
            pl.BlockSpec((1, D), lambda j: (0, 0)),
            pl.BlockSpec((1, D), lambda j: (0, 0)),
            pl.BlockSpec((1, D), lambda j: (0, 0)),
        ],
        out_specs=pl.BlockSpec((blk, D), lambda j: (j, 0)),
        out_shape=jax.ShapeDtypeStruct((NPAD, D), jnp.float32),
    )(S, x_pad, wr_sum, bias, gamma, beta)


@jax.jit
def kernel(x, edge_index, edge_type, W_l, b_l, W_r, emb, gamma, beta):
    z = _tc_pretransform(x, W_l)
    gkey = _tc_gkey(edge_index[0], edge_type)
    w = _sc_counts(edge_index[1], edge_type)
    S = _sc_aggregate(z, gkey, edge_index[1], w)
    wr_sum = jnp.sum(W_r, axis=0)
    bias = (jnp.sum(b_l, axis=0) + jnp.sum(emb, axis=0)).reshape(1, D)
    x_pad = jnp.pad(x, ((0, NPAD - N), (0, 0)))
    out = _tc_finish(S, x_pad, wr_sum, bias,
                     gamma.reshape(1, D), beta.reshape(1, D))
    return out[:N]
